# Initial kernel scaffold; baseline (speedup 1.0000x reference)
import functools

import jax
import jax.numpy as jnp
from jax import lax
from jax.experimental import pallas as pl
from jax.experimental.pallas import tpu as pltpu
from jax.experimental.pallas import tpu_sc as plsc

N = 10000
E = 320000
DIN = 128
H = 64
DOUT = 64
K = 10
ALPHA = 0.1

_INFO = plsc.get_sparse_core_info()
NC = _INFO.num_cores
NS = _INFO.num_subcores
NW = NC * NS
CHUNK = 128
CPW = -(-E // (NW * CHUNK))
EPAD = NW * CPW * CHUNK
RPT = -(-N // NS)
NPAD = NS * RPT

_MESH = plsc.VectorSubcoreMesh(core_axis_name="c", subcore_axis_name="s")


def _nan2num(a):
    return jnp.nan_to_num(a, nan=0.0, posinf=1000000.0, neginf=-1000000.0)


@functools.partial(
    pl.kernel,
    out_type=jax.ShapeDtypeStruct((NC, NPAD, 16), jnp.float32),
    mesh=_MESH,
    scratch_types=[
        pltpu.VMEM((CPW, CHUNK), jnp.int32),
        pltpu.VMEM((CHUNK, 16), jnp.float32),
        pltpu.VMEM_SHARED((NPAD, 16), jnp.float32),
    ],
)
def _deg_kernel(colm, zeros16, ones16, acc_out, col_v, ones_v, acc_sh):
    cid = lax.axis_index("c")
    sid = lax.axis_index("s")
    wid = sid * NC + cid
    pltpu.sync_copy(zeros16, acc_sh.at[pl.ds(sid * RPT, RPT)])
    pltpu.sync_copy(colm.at[pl.ds(wid * CPW, CPW)], col_v)
    pltpu.sync_copy(ones16, ones_v)
    plsc.subcore_barrier()

    def body(c, carry):
        pltpu.sync_copy(ones_v, acc_sh.at[col_v.at[c]], add=True)
        return carry

    lax.fori_loop(0, CPW, body, 0)
    plsc.subcore_barrier()
    pltpu.sync_copy(acc_sh.at[pl.ds(sid * RPT, RPT)],
                    acc_out.at[cid, pl.ds(sid * RPT, RPT)])


@functools.partial(
    pl.kernel,
    out_type=jax.ShapeDtypeStruct((NC, NPAD, DOUT), jnp.float32),
    mesh=_MESH,
    scratch_types=[
        pltpu.VMEM((CPW, CHUNK), jnp.int32),
        pltpu.VMEM((CPW, CHUNK), jnp.int32),
        pltpu.VMEM((CHUNK, DOUT), jnp.float32),
        pltpu.VMEM_SHARED((NPAD, DOUT), jnp.float32),
        pltpu.SemaphoreType.DMA,
    ],
)
def _scatter_kernel(s_hbm, rowm, colm, zeros64, agg_out,
                    row_v, col_v, buf, agg_sh, sem):
    cid = lax.axis_index("c")
    sid = lax.axis_index("s")
    wid = sid * NC + cid
    pltpu.sync_copy(zeros64, agg_sh.at[pl.ds(sid * RPT, RPT)])
    pltpu.sync_copy(rowm.at[pl.ds(wid * CPW, CPW)], row_v)
    pltpu.sync_copy(colm.at[pl.ds(wid * CPW, CPW)], col_v)
    plsc.subcore_barrier()

    def body(c, carry):
        pltpu.async_copy(s_hbm.at[row_v.at[c]], buf, sem).wait()
        pltpu.sync_copy(buf, agg_sh.at[col_v.at[c]], add=True)
        return carry

    lax.fori_loop(0, CPW, body, 0)
    plsc.subcore_barrier()
    pltpu.sync_copy(agg_sh.at[pl.ds(sid * RPT, RPT)],
                    agg_out.at[cid, pl.ds(sid * RPT, RPT)])


def _mlp_body(x_ref, w1_ref, b1_ref, w2_ref, b2_ref, acc_ref,
              h_ref, dinv_ref, s0_ref):
    x = _nan2num(x_ref[...])
    h1 = lax.dot_general(x, w1_ref[...], (((1,), (1,)), ((), ())),
                         preferred_element_type=jnp.float32)
    h1 = jnp.maximum(h1 + b1_ref[...], 0.0)
    h1 = _nan2num(h1)
    h = lax.dot_general(h1, w2_ref[...], (((1,), (1,)), ((), ())),
                        preferred_element_type=jnp.float32)
    h = _nan2num(h + b2_ref[...])
    deg = acc_ref[0, :, 0:1] + acc_ref[1, :, 0:1] + 1.0
    rows = lax.broadcasted_iota(jnp.int32, (NPAD, 1), 0)
    dinv = jnp.where(rows < N, lax.rsqrt(deg), 0.0)
    dinvb = jnp.broadcast_to(dinv, (NPAD, DOUT))
    h_ref[...] = h
    dinv_ref[...] = dinvb
    s0_ref[...] = dinvb * h


_mlp_call = pl.pallas_call(
    _mlp_body,
    out_shape=[
        jax.ShapeDtypeStruct((NPAD, DOUT), jnp.float32),
        jax.ShapeDtypeStruct((NPAD, DOUT), jnp.float32),
        jax.ShapeDtypeStruct((NPAD, DOUT), jnp.float32),
    ],
)


def _combine_body(agg_ref, s_ref, dinv_ref, h_ref, snew_ref, out_ref):
    dinvb = dinv_ref[...]
    t = (1.0 - ALPHA) * dinvb * (agg_ref[0] + agg_ref[1] + s_ref[...])
    t = t + ALPHA * h_ref[...]
    snew_ref[...] = dinvb * t
    out_ref[...] = _nan2num(t)


_combine_call = pl.pallas_call(
    _combine_body,
    out_shape=[
        jax.ShapeDtypeStruct((NPAD, DOUT), jnp.float32),
        jax.ShapeDtypeStruct((NPAD, DOUT), jnp.float32),
    ],
)


def kernel(x, edge_index, W1, b1, W2, b2):
    row = edge_index[0]
    col = edge_index[1]
    pad = EPAD - E
    rowp = jnp.concatenate([row, jnp.zeros((pad,), jnp.int32)]).reshape(NW * CPW, CHUNK)
    colp = jnp.concatenate([col, jnp.full((pad,), N, jnp.int32)]).reshape(NW * CPW, CHUNK)
    zeros16 = jnp.zeros((RPT, 16), jnp.float32)
    ones16 = jnp.ones((CHUNK, 16), jnp.float32)
    zeros64 = jnp.zeros((RPT, DOUT), jnp.float32)
    xp = jnp.pad(x, ((0, NPAD - N), (0, 0)))

    acc = _deg_kernel(colp, zeros16, ones16)
    h, dinvb, s = _mlp_call(xp, W1, b1.reshape(1, H), W2, b2.reshape(1, DOUT), acc)
    outv = h
    for _ in range(K):
        agg = _scatter_kernel(s, rowp, colp, zeros64)
        s, outv = _combine_call(agg, s, dinvb, h)
    return outv[:N]


# speedup vs baseline: 8.4327x; 8.4327x over previous
import functools

import jax
import jax.numpy as jnp
from jax import lax
from jax.experimental import pallas as pl
from jax.experimental.pallas import tpu as pltpu
from jax.experimental.pallas import tpu_sc as plsc

N = 10000
E = 320000
DIN = 128
H = 64
DOUT = 64
K = 10
ALPHA = 0.1

_INFO = plsc.get_sparse_core_info()
NC = _INFO.num_cores
NS = _INFO.num_subcores
NW = NC * NS
CHUNK = 128
CPW = -(-(-(-E // (NW * CHUNK))) // 8) * 8
EPAD = NW * CPW * CHUNK
RPT = -(-(-(-N // NS)) // 8) * 8
NPAD = NS * RPT

_MESH = plsc.VectorSubcoreMesh(core_axis_name="c", subcore_axis_name="s")
_SC_PARAMS = pltpu.CompilerParams(use_tc_tiling_on_sc=False)


def _nan2num(a):
    return jnp.nan_to_num(a, nan=0.0, posinf=1000000.0, neginf=-1000000.0)


@functools.partial(
    pl.kernel,
    out_type=jax.ShapeDtypeStruct((NC, NPAD, 16), jnp.float32),
    mesh=_MESH,
    scratch_types=[
        pltpu.VMEM((CPW, CHUNK), jnp.int32),
        pltpu.VMEM((CHUNK, 16), jnp.float32),
        pltpu.VMEM_SHARED((NPAD, 16), jnp.float32),
    ],
    compiler_params=_SC_PARAMS,
)
def _deg_kernel(colm, zeros16, ones16, acc_out, col_v, ones_v, acc_sh):
    cid = lax.axis_index("c")
    sid = lax.axis_index("s")
    wid = sid * NC + cid
    pltpu.sync_copy(zeros16, acc_sh.at[pl.ds(sid * RPT, RPT)])
    pltpu.sync_copy(colm.at[pl.ds(wid * CPW, CPW)], col_v)
    pltpu.sync_copy(ones16, ones_v)
    plsc.subcore_barrier()

    def body(c, carry):
        pltpu.sync_copy(ones_v, acc_sh.at[col_v.at[c]], add=True)
        return carry

    lax.fori_loop(0, CPW, body, 0)
    plsc.subcore_barrier()
    pltpu.sync_copy(acc_sh.at[pl.ds(sid * RPT, RPT)],
                    acc_out.at[cid, pl.ds(sid * RPT, RPT)])


@functools.partial(
    pl.kernel,
    out_type=jax.ShapeDtypeStruct((NC, NPAD, DOUT), jnp.float32),
    mesh=_MESH,
    scratch_types=[
        pltpu.VMEM((CPW, CHUNK), jnp.int32),
        pltpu.VMEM((CPW, CHUNK), jnp.int32),
        pltpu.VMEM((CHUNK, DOUT), jnp.float32),
        pltpu.VMEM_SHARED((NPAD, DOUT), jnp.float32),
        pltpu.SemaphoreType.DMA,
    ],
    compiler_params=_SC_PARAMS,
)
def _scatter_kernel(s_hbm, rowm, colm, zeros64, agg_out,
                    row_v, col_v, buf, agg_sh, sem):
    cid = lax.axis_index("c")
    sid = lax.axis_index("s")
    wid = sid * NC + cid
    pltpu.sync_copy(zeros64, agg_sh.at[pl.ds(sid * RPT, RPT)])
    pltpu.sync_copy(rowm.at[pl.ds(wid * CPW, CPW)], row_v)
    pltpu.sync_copy(colm.at[pl.ds(wid * CPW, CPW)], col_v)
    plsc.subcore_barrier()

    def body(c, carry):
        pltpu.async_copy(s_hbm.at[row_v.at[c]], buf, sem).wait()
        pltpu.sync_copy(buf, agg_sh.at[col_v.at[c]], add=True)
        return carry

    lax.fori_loop(0, CPW, body, 0)
    plsc.subcore_barrier()
    pltpu.sync_copy(agg_sh.at[pl.ds(sid * RPT, RPT)],
                    agg_out.at[cid, pl.ds(sid * RPT, RPT)])


def _mlp_body(x_ref, w1_ref, b1_ref, w2_ref, b2_ref, acc_ref,
              h_ref, dinv_ref, s0_ref):
    x = _nan2num(x_ref[...])
    h1 = lax.dot_general(x, w1_ref[...], (((1,), (1,)), ((), ())),
                         preferred_element_type=jnp.float32)
    h1 = jnp.maximum(h1 + b1_ref[...], 0.0)
    h1 = _nan2num(h1)
    h = lax.dot_general(h1, w2_ref[...], (((1,), (1,)), ((), ())),
                        preferred_element_type=jnp.float32)
    h = _nan2num(h + b2_ref[...])
    deg = acc_ref[0, :, 0:1] + acc_ref[1, :, 0:1] + 1.0
    rows = lax.broadcasted_iota(jnp.int32, (NPAD, 1), 0)
    dinv = jnp.where(rows < N, lax.rsqrt(deg), 0.0)
    dinvb = jnp.broadcast_to(dinv, (NPAD, DOUT))
    h_ref[...] = h
    dinv_ref[...] = dinvb
    s0_ref[...] = dinvb * h


_mlp_call = pl.pallas_call(
    _mlp_body,
    out_shape=[
        jax.ShapeDtypeStruct((NPAD, DOUT), jnp.float32),
        jax.ShapeDtypeStruct((NPAD, DOUT), jnp.float32),
        jax.ShapeDtypeStruct((NPAD, DOUT), jnp.float32),
    ],
)


def _combine_body(agg_ref, s_ref, dinv_ref, h_ref, snew_ref, out_ref):
    dinvb = dinv_ref[...]
    t = (1.0 - ALPHA) * dinvb * (agg_ref[0] + agg_ref[1] + s_ref[...])
    t = t + ALPHA * h_ref[...]
    snew_ref[...] = dinvb * t
    out_ref[...] = _nan2num(t)


_combine_call = pl.pallas_call(
    _combine_body,
    out_shape=[
        jax.ShapeDtypeStruct((NPAD, DOUT), jnp.float32),
        jax.ShapeDtypeStruct((NPAD, DOUT), jnp.float32),
    ],
)


def kernel(x, edge_index, W1, b1, W2, b2):
    row = edge_index[0]
    col = edge_index[1]
    pad = EPAD - E
    rowp = jnp.concatenate([row, jnp.zeros((pad,), jnp.int32)]).reshape(NW * CPW, CHUNK)
    colp = jnp.concatenate([col, jnp.full((pad,), N, jnp.int32)]).reshape(NW * CPW, CHUNK)
    zeros16 = jnp.zeros((RPT, 16), jnp.float32)
    ones16 = jnp.ones((CHUNK, 16), jnp.float32)
    zeros64 = jnp.zeros((RPT, DOUT), jnp.float32)
    xp = jnp.pad(x, ((0, NPAD - N), (0, 0)))

    acc = _deg_kernel(colp, zeros16, ones16)
    h, dinvb, s = _mlp_call(xp, W1, b1.reshape(1, H), W2, b2.reshape(1, DOUT), acc)
    outv = h
    for _ in range(K):
        agg = _scatter_kernel(s, rowp, colp, zeros64)
        s, outv = _combine_call(agg, s, dinvb, h)
    return outv[:N]


# speedup vs baseline: 9.7184x; 1.1525x over previous
import functools

import jax
import jax.numpy as jnp
from jax import lax
from jax.experimental import pallas as pl
from jax.experimental.pallas import tpu as pltpu
from jax.experimental.pallas import tpu_sc as plsc

N = 10000
E = 320000
DIN = 128
H = 64
DOUT = 64
K = 10
ALPHA = 0.1

_INFO = plsc.get_sparse_core_info()
NC = _INFO.num_cores
NS = _INFO.num_subcores
NW = NC * NS
CHUNK = 128
CPW = -(-(-(-E // (NW * CHUNK))) // 8) * 8
EPAD = NW * CPW * CHUNK
RPT = -(-(-(-N // NS)) // 8) * 8
NPAD = NS * RPT

_MESH = plsc.VectorSubcoreMesh(core_axis_name="c", subcore_axis_name="s")
_SC_PARAMS = pltpu.CompilerParams(use_tc_tiling_on_sc=False)


def _nan2num(a):
    return jnp.nan_to_num(a, nan=0.0, posinf=1000000.0, neginf=-1000000.0)


@functools.partial(
    pl.kernel,
    out_type=jax.ShapeDtypeStruct((NC, NPAD, 16), jnp.float32),
    mesh=_MESH,
    scratch_types=[
        pltpu.VMEM((CPW, CHUNK), jnp.int32),
        pltpu.VMEM((CHUNK, 16), jnp.float32),
        pltpu.VMEM_SHARED((NPAD, 16), jnp.float32),
        pltpu.SemaphoreType.DMA,
    ],
    compiler_params=_SC_PARAMS,
)
def _deg_kernel(colm, zeros16, ones16, acc_out, col_v, ones_v, acc_sh, sem):
    cid = lax.axis_index("c")
    sid = lax.axis_index("s")
    wid = sid * NC + cid
    pltpu.sync_copy(zeros16, acc_sh.at[pl.ds(sid * RPT, RPT)])
    pltpu.sync_copy(colm.at[pl.ds(wid * CPW, CPW)], col_v)
    pltpu.sync_copy(ones16, ones_v)
    plsc.subcore_barrier()

    def body(c, carry):
        pltpu.async_copy(ones_v, acc_sh.at[col_v.at[c]], sem, add=True)
        return carry

    lax.fori_loop(0, CPW, body, 0)

    def drain(c, carry):
        pltpu.make_async_copy(ones_v, acc_sh.at[col_v.at[c]], sem).wait()
        return carry

    lax.fori_loop(0, CPW, drain, 0)
    plsc.subcore_barrier()
    pltpu.sync_copy(acc_sh.at[pl.ds(sid * RPT, RPT)],
                    acc_out.at[cid, pl.ds(sid * RPT, RPT)])


NBUF = 8
GROUPS = CPW // NBUF


@functools.partial(
    pl.kernel,
    out_type=jax.ShapeDtypeStruct((NC, NPAD, DOUT), jnp.float32),
    mesh=_MESH,
    scratch_types=(
        [pltpu.VMEM((CPW, CHUNK), jnp.int32),
         pltpu.VMEM((CPW, CHUNK), jnp.int32)]
        + [pltpu.VMEM((CHUNK, DOUT), jnp.float32) for _ in range(NBUF)]
        + [pltpu.VMEM_SHARED((NPAD, DOUT), jnp.float32)]
        + [pltpu.SemaphoreType.DMA for _ in range(2 * NBUF)]
    ),
    compiler_params=_SC_PARAMS,
)
def _scatter_kernel(s_hbm, rowm, colm, zeros64, agg_out, row_v, col_v, *rest):
    bufs = rest[:NBUF]
    agg_sh = rest[NBUF]
    gsem = rest[NBUF + 1:NBUF + 1 + NBUF]
    ssem = rest[NBUF + 1 + NBUF:]
    cid = lax.axis_index("c")
    sid = lax.axis_index("s")
    wid = sid * NC + cid
    pltpu.sync_copy(zeros64, agg_sh.at[pl.ds(sid * RPT, RPT)])
    pltpu.sync_copy(rowm.at[pl.ds(wid * CPW, CPW)], row_v)
    pltpu.sync_copy(colm.at[pl.ds(wid * CPW, CPW)], col_v)
    plsc.subcore_barrier()

    def fire_gather(b, c):
        pltpu.async_copy(s_hbm.at[row_v.at[c]], bufs[b], gsem[b])

    def wait_gather(b, c):
        pltpu.make_async_copy(s_hbm.at[row_v.at[c]], bufs[b], gsem[b]).wait()

    def fire_scat(b, c):
        pltpu.async_copy(bufs[b], agg_sh.at[col_v.at[c]], ssem[b], add=True)

    def wait_scat(b, c):
        pltpu.make_async_copy(bufs[b], agg_sh.at[col_v.at[c]], ssem[b]).wait()

    for b in range(NBUF):
        fire_gather(b, b)

    def outer(g, carry):
        base = g * NBUF
        for b in range(NBUF):
            wait_gather(b, base + b)
            fire_scat(b, base + b)
        for b in range(NBUF):
            wait_scat(b, base + b)
            fire_gather(b, base + NBUF + b)
        return carry

    lax.fori_loop(0, GROUPS - 1, outer, 0)
    last = (GROUPS - 1) * NBUF
    for b in range(NBUF):
        wait_gather(b, last + b)
        fire_scat(b, last + b)
    for b in range(NBUF):
        wait_scat(b, last + b)
    plsc.subcore_barrier()
    pltpu.sync_copy(agg_sh.at[pl.ds(sid * RPT, RPT)],
                    agg_out.at[cid, pl.ds(sid * RPT, RPT)])


def _mlp_body(x_ref, w1_ref, b1_ref, w2_ref, b2_ref, acc_ref,
              h_ref, dinv_ref, s0_ref):
    x = _nan2num(x_ref[...])
    h1 = lax.dot_general(x, w1_ref[...], (((1,), (1,)), ((), ())),
                         preferred_element_type=jnp.float32)
    h1 = jnp.maximum(h1 + b1_ref[...], 0.0)
    h1 = _nan2num(h1)
    h = lax.dot_general(h1, w2_ref[...], (((1,), (1,)), ((), ())),
                        preferred_element_type=jnp.float32)
    h = _nan2num(h + b2_ref[...])
    deg = acc_ref[0, :, 0:1] + acc_ref[1, :, 0:1] + 1.0
    rows = lax.broadcasted_iota(jnp.int32, (NPAD, 1), 0)
    dinv = jnp.where(rows < N, lax.rsqrt(deg), 0.0)
    dinvb = jnp.broadcast_to(dinv, (NPAD, DOUT))
    h_ref[...] = h
    dinv_ref[...] = dinvb
    s0_ref[...] = dinvb * h


_mlp_call = pl.pallas_call(
    _mlp_body,
    out_shape=[
        jax.ShapeDtypeStruct((NPAD, DOUT), jnp.float32),
        jax.ShapeDtypeStruct((NPAD, DOUT), jnp.float32),
        jax.ShapeDtypeStruct((NPAD, DOUT), jnp.float32),
    ],
)


def _combine_body(agg_ref, s_ref, dinv_ref, h_ref, snew_ref, out_ref):
    dinvb = dinv_ref[...]
    t = (1.0 - ALPHA) * dinvb * (agg_ref[0] + agg_ref[1] + s_ref[...])
    t = t + ALPHA * h_ref[...]
    snew_ref[...] = dinvb * t
    out_ref[...] = _nan2num(t)


_combine_call = pl.pallas_call(
    _combine_body,
    out_shape=[
        jax.ShapeDtypeStruct((NPAD, DOUT), jnp.float32),
        jax.ShapeDtypeStruct((NPAD, DOUT), jnp.float32),
    ],
)


def kernel(x, edge_index, W1, b1, W2, b2):
    row = edge_index[0]
    col = edge_index[1]
    pad = EPAD - E
    rowp = jnp.concatenate([row, jnp.zeros((pad,), jnp.int32)]).reshape(NW * CPW, CHUNK)
    colp = jnp.concatenate([col, jnp.full((pad,), N, jnp.int32)]).reshape(NW * CPW, CHUNK)
    zeros16 = jnp.zeros((RPT, 16), jnp.float32)
    ones16 = jnp.ones((CHUNK, 16), jnp.float32)
    zeros64 = jnp.zeros((RPT, DOUT), jnp.float32)
    xp = jnp.pad(x, ((0, NPAD - N), (0, 0)))

    acc = _deg_kernel(colp, zeros16, ones16)
    h, dinvb, s = _mlp_call(xp, W1, b1.reshape(1, H), W2, b2.reshape(1, DOUT), acc)
    outv = h
    for _ in range(K):
        agg = _scatter_kernel(s, rowp, colp, zeros64)
        s, outv = _combine_call(agg, s, dinvb, h)
    return outv[:N]
